# baseline (speedup 1.0000x reference)
import functools

import jax
import jax.numpy as jnp
from jax import lax
from jax.experimental import pallas as pl
from jax.experimental.pallas import tpu as pltpu
from jax.experimental.pallas import tpu_sc as plsc

N_NODES = 10000
N_EDGES = 320000
D = 128
D2 = 256

NC = 2
NS = 16
NW = NC * NS
E_PER_TILE = N_EDGES // NW
CHUNK = 125
N_CHUNKS = E_PER_TILE // CHUNK
IDX_BLK = 16
N_BLKS = N_CHUNKS // IDX_BLK
STRIPE = 632
STRIPE_LAST = N_NODES - 15 * STRIPE

_MESH = plsc.VectorSubcoreMesh(core_axis_name="c", subcore_axis_name="s")


def _zero_stripe(sid, zrows, acc_sh):
    @pl.when(sid < NS - 1)
    def _():
        pltpu.sync_copy(zrows.at[pl.ds(sid * STRIPE, STRIPE)],
                        acc_sh.at[pl.ds(sid * STRIPE, STRIPE)])

    @pl.when(sid == NS - 1)
    def _():
        pltpu.sync_copy(zrows.at[pl.ds(15 * STRIPE, STRIPE_LAST)],
                        acc_sh.at[pl.ds(15 * STRIPE, STRIPE_LAST)])


def _copyout_stripe(sid, cid, acc_sh, out_hbm):
    @pl.when(sid < NS - 1)
    def _():
        pltpu.sync_copy(acc_sh.at[pl.ds(sid * STRIPE, STRIPE)],
                        out_hbm.at[cid, pl.ds(sid * STRIPE, STRIPE)])

    @pl.when(sid == NS - 1)
    def _():
        pltpu.sync_copy(acc_sh.at[pl.ds(15 * STRIPE, STRIPE_LAST)],
                        out_hbm.at[cid, pl.ds(15 * STRIPE, STRIPE_LAST)])


def _sc_body(with_cnt, y, srcs, dsts, zrows, *rest):
    if with_cnt:
        (ones_h, agg_out, cnt_out,
         src_v, dst_v, rows_v, acc_sh, sem) = rest
    else:
        agg_out, src_v, dst_v, rows_v, acc_sh, sem = rest
    cid = lax.axis_index("c")
    sid = lax.axis_index("s")
    pltpu.sync_copy(srcs.at[cid, sid, pl.ds(0, IDX_BLK)], src_v)
    pltpu.sync_copy(dsts.at[cid, sid, pl.ds(0, IDX_BLK)], dst_v)
    pltpu.async_copy(y.at[src_v.at[0]], rows_v.at[0], sem)
    _zero_stripe(sid, zrows, acc_sh)
    plsc.subcore_barrier()

    def step_g(g, carry):
        @pl.when(g > 0)
        def _():
            pltpu.sync_copy(srcs.at[cid, sid, pl.ds(g * IDX_BLK, IDX_BLK)],
                            src_v)
            pltpu.sync_copy(dsts.at[cid, sid, pl.ds(g * IDX_BLK, IDX_BLK)],
                            dst_v)
            pltpu.async_copy(y.at[src_v.at[0]], rows_v.at[0], sem)
        for j in range(IDX_BLK):
            pltpu.make_async_copy(y.at[src_v.at[j]], rows_v.at[j % 2],
                                  sem).wait()
            if j + 1 < IDX_BLK:
                pltpu.async_copy(y.at[src_v.at[j + 1]],
                                 rows_v.at[(j + 1) % 2], sem)
            pltpu.sync_copy(rows_v.at[j % 2], acc_sh.at[dst_v.at[j]],
                            add=True)
        return carry

    lax.fori_loop(0, N_BLKS, step_g, 0)

    plsc.subcore_barrier()
    _copyout_stripe(sid, cid, acc_sh, agg_out)

    if with_cnt:
        ones_v = rows_v.at[0]
        pltpu.sync_copy(ones_h, ones_v)
        _zero_stripe(sid, zrows, acc_sh)
        plsc.subcore_barrier()

        def cnt_g(g, carry):
            pltpu.sync_copy(dsts.at[cid, sid, pl.ds(g * IDX_BLK, IDX_BLK)],
                            dst_v)
            for j in range(IDX_BLK):
                pltpu.async_copy(ones_v, acc_sh.at[dst_v.at[j]], sem,
                                 add=True)
            for j in range(IDX_BLK):
                pltpu.make_async_copy(ones_v, acc_sh.at[dst_v.at[j]],
                                      sem).wait()
            return carry

        lax.fori_loop(0, N_BLKS, cnt_g, 0)
        plsc.subcore_barrier()
        _copyout_stripe(sid, cid, acc_sh, cnt_out)


_sc_agg_cnt = pl.kernel(
    functools.partial(_sc_body, True),
    mesh=_MESH,
    out_type=[
        jax.ShapeDtypeStruct((NC, N_NODES, D), jnp.float32),
        jax.ShapeDtypeStruct((NC, N_NODES, D), jnp.float32),
    ],
    scratch_types=[
        pltpu.VMEM((IDX_BLK, CHUNK), jnp.int32),
        pltpu.VMEM((IDX_BLK, CHUNK), jnp.int32),
        pltpu.VMEM((2, CHUNK, D), jnp.float32),
        pltpu.VMEM_SHARED((N_NODES, D), jnp.float32),
        pltpu.SemaphoreType.DMA,
    ],
)

_sc_agg = pl.kernel(
    functools.partial(_sc_body, False),
    mesh=_MESH,
    out_type=jax.ShapeDtypeStruct((NC, N_NODES, D), jnp.float32),
    scratch_types=[
        pltpu.VMEM((IDX_BLK, CHUNK), jnp.int32),
        pltpu.VMEM((IDX_BLK, CHUNK), jnp.int32),
        pltpu.VMEM((2, CHUNK, D), jnp.float32),
        pltpu.VMEM_SHARED((N_NODES, D), jnp.float32),
        pltpu.SemaphoreType.DMA,
    ],
)

BLK = 2000


def _tc_layer1(agg, cnt, x, Wl, Wr, b):
    def body(a_ref, c_ref, x_ref, wl_ref, wr_ref, b_ref, o_ref):
        c = jnp.maximum(c_ref[0, :, 0:1] + c_ref[1, :, 0:1], 1.0)
        mean = (a_ref[0] + a_ref[1]) / c
        o_ref[...] = jnp.maximum(
            jnp.dot(mean, wl_ref[...], preferred_element_type=jnp.float32)
            + jnp.dot(x_ref[...], wr_ref[...], preferred_element_type=jnp.float32)
            + b_ref[...],
            0.0,
        )

    return pl.pallas_call(
        body,
        grid=(N_NODES // BLK,),
        in_specs=[
            pl.BlockSpec((NC, BLK, D), lambda i: (0, i, 0)),
            pl.BlockSpec((NC, BLK, D), lambda i: (0, i, 0)),
            pl.BlockSpec((BLK, D), lambda i: (i, 0)),
            pl.BlockSpec((D, D), lambda i: (0, 0)),
            pl.BlockSpec((D, D), lambda i: (0, 0)),
            pl.BlockSpec((1, D), lambda i: (0, 0)),
        ],
        out_specs=pl.BlockSpec((BLK, D), lambda i: (i, 0)),
        out_shape=jax.ShapeDtypeStruct((N_NODES, D), jnp.float32),
    )(agg, cnt, x, Wl, Wr, b)


def _tc_final(agg2, cnt, h1, x, Wl2, Wr2, b2, Wlin, blin, Wsc, bsc):
    def body(a_ref, c_ref, h1_ref, x_ref, wl2_ref, wr2_ref, b2_ref,
             wlin_ref, blin_ref, wsc_ref, bsc_ref, o_ref):
        c = jnp.maximum(c_ref[0, :, 0:1] + c_ref[1, :, 0:1], 1.0)
        mean = (a_ref[0] + a_ref[1]) / c
        h2 = jnp.maximum(
            jnp.dot(mean, wl2_ref[...], preferred_element_type=jnp.float32)
            + jnp.dot(h1_ref[...], wr2_ref[...], preferred_element_type=jnp.float32)
            + b2_ref[...],
            0.0,
        )
        o_ref[...] = (
            jnp.dot(h2, wlin_ref[...], preferred_element_type=jnp.float32)
            + blin_ref[...]
            + jnp.dot(x_ref[...], wsc_ref[...], preferred_element_type=jnp.float32)
            + bsc_ref[...]
        )

    return pl.pallas_call(
        body,
        grid=(N_NODES // BLK,),
        in_specs=[
            pl.BlockSpec((NC, BLK, D), lambda i: (0, i, 0)),
            pl.BlockSpec((NC, BLK, D), lambda i: (0, i, 0)),
            pl.BlockSpec((BLK, D), lambda i: (i, 0)),
            pl.BlockSpec((BLK, D), lambda i: (i, 0)),
            pl.BlockSpec((D, D2), lambda i: (0, 0)),
            pl.BlockSpec((D, D2), lambda i: (0, 0)),
            pl.BlockSpec((1, D2), lambda i: (0, 0)),
            pl.BlockSpec((D2, D), lambda i: (0, 0)),
            pl.BlockSpec((1, D), lambda i: (0, 0)),
            pl.BlockSpec((D, D), lambda i: (0, 0)),
            pl.BlockSpec((1, D), lambda i: (0, 0)),
        ],
        out_specs=pl.BlockSpec((BLK, D), lambda i: (i, 0)),
        out_shape=jax.ShapeDtypeStruct((N_NODES, D), jnp.float32),
    )(agg2, cnt, h1, x, Wl2, Wr2, b2, Wlin, blin, Wsc, bsc)


def kernel(x, edge_index, Wl1, Wr1, b1, Wl2, Wr2, b2, Wlin, blin, Wsc, bsc):
    ei = edge_index.astype(jnp.int32)
    srcs = ei[0].reshape(NC, NS, N_CHUNKS, CHUNK)
    dsts = ei[1].reshape(NC, NS, N_CHUNKS, CHUNK)
    del ei
    zrows = jnp.zeros((N_NODES, D), jnp.float32)
    ones_h = jnp.ones((CHUNK, D), jnp.float32)

    agg1, cnt = _sc_agg_cnt(x, srcs, dsts, zrows, ones_h)
    h1 = _tc_layer1(agg1, cnt, x, Wl1, Wr1, b1.reshape(1, D))
    agg2 = _sc_agg(h1, srcs, dsts, zrows)
    out = _tc_final(agg2, cnt, h1, x, Wl2, Wr2, b2.reshape(1, D2),
                    Wlin, blin.reshape(1, D), Wsc, bsc.reshape(1, D))
    return out
